# baseline (speedup 1.0000x reference)
import functools
import math

import jax
import jax.numpy as jnp
from jax import lax
from jax.experimental import pallas as pl
from jax.experimental.pallas import tpu as pltpu
from jax.experimental.pallas import tpu_sc as plsc

VOCAB = 100000
D = 64
MAX_LEN = 1024
LANES = 16
NC, NS = 2, 16
NW = NC * NS
ROWS = 1024 * 1024
ROWS_PER_W = ROWS // NW
CHUNK = 128
N_CHUNKS = ROWS_PER_W // CHUNK
PE_WINDOWS = MAX_LEN // CHUNK


def _sin_pe(max_len, d_model):
    pos = jnp.arange(0, max_len, dtype=jnp.float32)[:, None]
    div = jnp.exp(jnp.arange(0, d_model, 2, dtype=jnp.float32)
                  * (-(math.log(10000.0) / d_model)))
    pe = jnp.zeros((max_len, d_model), dtype=jnp.float32)
    pe = pe.at[:, 0::2].set(jnp.sin(pos * div))
    pe = pe.at[:, 1::2].set(jnp.cos(pos * div))
    return pe


def _sc_body(table_hbm, ids_hbm, pe_hbm, out_hbm,
             idx_v, rows0, rows1, pe_v, gsem, osem):
    wid = lax.axis_index("s") * NC + lax.axis_index("c")
    base0 = wid * ROWS_PER_W
    pltpu.sync_copy(ids_hbm.at[pl.ds(wid * N_CHUNKS, N_CHUNKS)], idx_v)
    pltpu.sync_copy(pe_hbm, pe_v)

    def g_issue(c, buf):
        pltpu.async_copy(table_hbm.at[idx_v.at[c]], buf, gsem)

    def g_wait(buf):
        pltpu.make_async_copy(table_hbm.at[idx_v.at[0]], buf, gsem).wait()

    def s_issue(c, buf):
        pltpu.async_copy(buf, out_hbm.at[pl.ds(base0 + c * CHUNK, CHUNK)],
                         osem)

    def s_wait(buf):
        pltpu.make_async_copy(buf, out_hbm.at[pl.ds(0, CHUNK)], osem).wait()

    def add_pe(buf, p0):
        def _row(i, carry2):
            for j in range(D // LANES):
                v = pe_v[p0 + i, pl.ds(j * LANES, LANES)]
                plsc.addupdate(buf.at[i, pl.ds(j * LANES, LANES)], v)
            return carry2

        lax.fori_loop(0, CHUNK, _row, 0, unroll=4)

    g_issue(0, rows0)
    g_wait(rows0)
    g_issue(1, rows1)
    add_pe(rows0, 0)
    s_issue(0, rows0)
    g_wait(rows1)
    s_wait(rows0)
    g_issue(2, rows0)
    add_pe(rows1, CHUNK)
    s_issue(1, rows1)

    def body(C, carry):
        e = 2 * C
        p0 = lax.rem(C, PE_WINDOWS // 2) * (2 * CHUNK)
        g_wait(rows0)
        s_wait(rows1)
        g_issue(e + 1, rows1)
        add_pe(rows0, p0)
        s_issue(e, rows0)
        g_wait(rows1)
        s_wait(rows0)

        @pl.when(C < N_CHUNKS // 2 - 1)
        def _():
            g_issue(e + 2, rows0)

        add_pe(rows1, p0 + CHUNK)
        s_issue(e + 1, rows1)
        return carry

    lax.fori_loop(1, N_CHUNKS // 2, body, 0)
    s_wait(rows1)


@jax.jit
def _tpe_sc(ids_flat2d, table, pe):
    mesh = plsc.VectorSubcoreMesh(core_axis_name="c", subcore_axis_name="s")
    k = functools.partial(
        pl.kernel,
        out_type=jax.ShapeDtypeStruct((ROWS, D), jnp.float32),
        mesh=mesh,
        scratch_types=[
            pltpu.VMEM((N_CHUNKS, CHUNK), jnp.int32),
            pltpu.VMEM((CHUNK, D), jnp.float32),
            pltpu.VMEM((CHUNK, D), jnp.float32),
            pltpu.VMEM((MAX_LEN, D), jnp.float32),
            pltpu.SemaphoreType.DMA,
            pltpu.SemaphoreType.DMA,
        ],
        compiler_params=pltpu.CompilerParams(use_tc_tiling_on_sc=False),
    )(_sc_body)
    return k(table, ids_flat2d, pe)


def kernel(input_ids, table):
    b, s = input_ids.shape
    ids = input_ids.reshape(ROWS // CHUNK, CHUNK).astype(jnp.int32)
    pe = _sin_pe(MAX_LEN, D)
    out = _tpe_sc(ids, table, pe)
    return out.reshape(b, s, D)
